# baseline (speedup 1.0000x reference)
import functools
import math

import jax
import jax.numpy as jnp
from jax import lax
from jax.experimental import pallas as pl
from jax.experimental.pallas import tpu as pltpu
from jax.experimental.pallas import tpu_sc as plsc

D = 64
NC, NS, L = 2, 16, 16
NW = NC * NS


def kernel(x, table, pe):
    B, S = x.shape
    N = B * S
    n_per_w = N // NW
    C = 128
    n_chunks = n_per_w // C
    spc = S // C

    x1 = x.reshape(N).astype(jnp.int32)
    pe_t = pe.reshape(S, D).astype(jnp.float32).T
    table128 = jnp.pad(table, ((0, 0), (0, 128 - D)))
    scale = jnp.float32(math.sqrt(D))

    mesh = plsc.VectorSubcoreMesh(
        core_axis_name="c", subcore_axis_name="s",
        num_cores=NC, num_subcores=NS)

    @functools.partial(
        pl.kernel,
        out_type=jax.ShapeDtypeStruct((B, D, S), jnp.float32),
        mesh=mesh,
        compiler_params=pltpu.CompilerParams(needs_layout_passes=False),
        scratch_types=[
            pltpu.VMEM((D, S), jnp.float32),
            pltpu.VMEM((n_per_w,), jnp.int32),
            pltpu.VMEM((C, 128), jnp.float32),
            pltpu.VMEM((C, 128), jnp.float32),
            pltpu.VMEM((D, C), jnp.float32),
            pltpu.VMEM((D, C), jnp.float32),
            pltpu.SemaphoreType.DMA,
            pltpu.SemaphoreType.DMA,
            pltpu.SemaphoreType.DMA,
            pltpu.SemaphoreType.DMA,
        ],
    )
    def body(x_hbm, table_hbm, pe_hbm, out_hbm, pe_v, idx_v,
             r0, r1, o0, o1, gs0, gs1, ss0, ss1):
        wid = lax.axis_index("s") * NC + lax.axis_index("c")
        w_base = wid * n_per_w
        w_seq = wid * (n_per_w // S)
        pltpu.sync_copy(pe_hbm, pe_v)
        pltpu.sync_copy(x_hbm.at[pl.ds(w_base, n_per_w)], idx_v)

        def fire_gather(g, rbuf, gsem):
            pltpu.async_copy(
                table_hbm.at[idx_v.at[pl.ds(g * C, C)]], rbuf, gsem)

        def drain_gather(rbuf, gsem):
            pltpu.make_async_copy(
                table_hbm.at[pl.ds(0, C)], rbuf, gsem).wait()

        def store_dst(g):
            return out_hbm.at[w_seq + lax.div(g, spc), :,
                              pl.ds(lax.rem(g, spc) * C, C)]

        def compute(g, rbuf, obuf):
            po = lax.rem(g, spc) * C

            @pl.loop(0, C // L)
            def sblk_loop(sb):
                svec = lax.iota(jnp.int32, L) + sb * L

                @plsc.parallel_loop(0, D, unroll=4)
                def d_loop(d):
                    dvec = jnp.full((L,), d, jnp.int32)
                    v = plsc.load_gather(rbuf, [svec, dvec])
                    obuf[d, pl.ds(sb * L, L)] = (
                        v * scale + pe_v[d, pl.ds(po + sb * L, L)])

        fire_gather(0, r0, gs0)

        @pl.loop(0, n_chunks // 2)
        def pair_loop(gg):
            g = gg * 2
            fire_gather(g + 1, r1, gs1)
            drain_gather(r0, gs0)

            @pl.when(gg > 0)
            def _wait_store0():
                pltpu.make_async_copy(
                    o0, out_hbm.at[0, :, pl.ds(0, C)], ss0).wait()
            compute(g, r0, o0)
            pltpu.async_copy(o0, store_dst(g), ss0)

            @pl.when(gg + 1 < n_chunks // 2)
            def _fire_next():
                fire_gather(g + 2, r0, gs0)
            drain_gather(r1, gs1)

            @pl.when(gg > 0)
            def _wait_store1():
                pltpu.make_async_copy(
                    o1, out_hbm.at[0, :, pl.ds(0, C)], ss1).wait()
            compute(g + 1, r1, o1)
            pltpu.async_copy(o1, store_dst(g + 1), ss1)

        pltpu.make_async_copy(o0, out_hbm.at[0, :, pl.ds(0, C)], ss0).wait()
        pltpu.make_async_copy(o1, out_hbm.at[0, :, pl.ds(0, C)], ss1).wait()

    out = body(x1, table128, pe_t)
    return out.transpose(0, 2, 1)


# speedup vs baseline: 1.5376x; 1.5376x over previous
import functools
import math

import jax
import jax.numpy as jnp
from jax import lax
from jax.experimental import pallas as pl
from jax.experimental.pallas import tpu as pltpu
from jax.experimental.pallas import tpu_sc as plsc

D = 64
NC, NS, L = 2, 16, 16
NW = NC * NS


def kernel(x, table, pe):
    B, S = x.shape
    N = B * S
    n_per_w = N // NW
    C = 128
    n_chunks = n_per_w // C
    spc = S // C

    x1 = x.reshape(N).astype(jnp.int32)
    pe1 = pe.astype(jnp.float32).reshape(S * D)
    table128 = jnp.pad(table, ((0, 0), (0, 128 - D)))
    scale = jnp.float32(math.sqrt(D))

    mesh = plsc.VectorSubcoreMesh(
        core_axis_name="c", subcore_axis_name="s",
        num_cores=NC, num_subcores=NS)

    @functools.partial(
        pl.kernel,
        out_type=jax.ShapeDtypeStruct((N, D), jnp.float32),
        mesh=mesh,
        compiler_params=pltpu.CompilerParams(needs_layout_passes=False),
        scratch_types=[
            pltpu.VMEM((S * D,), jnp.float32),
            pltpu.VMEM((n_per_w,), jnp.int32),
            pltpu.VMEM((C, 128), jnp.float32),
            pltpu.VMEM((C, 128), jnp.float32),
            pltpu.VMEM((C, D), jnp.float32),
            pltpu.VMEM((C, D), jnp.float32),
            pltpu.SemaphoreType.DMA,
            pltpu.SemaphoreType.DMA,
            pltpu.SemaphoreType.DMA,
            pltpu.SemaphoreType.DMA,
        ],
    )
    def body(x_hbm, table_hbm, pe_hbm, out_hbm, pe_v, idx_v,
             r0, r1, o0, o1, gs0, gs1, ss0, ss1):
        wid = lax.axis_index("s") * NC + lax.axis_index("c")
        w_base = wid * n_per_w
        pltpu.sync_copy(pe_hbm, pe_v)
        pltpu.sync_copy(x_hbm.at[pl.ds(w_base, n_per_w)], idx_v)

        def fire_gather(g, rbuf, gsem):
            pltpu.async_copy(
                table_hbm.at[idx_v.at[pl.ds(g * C, C)]], rbuf, gsem)

        def drain_gather(rbuf, gsem):
            pltpu.make_async_copy(
                table_hbm.at[pl.ds(0, C)], rbuf, gsem).wait()

        def drain_store(obuf, ssem):
            pltpu.make_async_copy(
                obuf, out_hbm.at[pl.ds(0, C)], ssem).wait()

        def compute(g, rbuf, obuf):
            po = lax.rem(g, spc) * (C * D)

            @plsc.parallel_loop(0, C, unroll=4)
            def row_loop(r):
                for k in range(0, D, L):
                    obuf[r, pl.ds(k, L)] = (
                        rbuf[r, pl.ds(k, L)] * scale
                        + pe_v[pl.ds(po + r * D + k, L)])

        fire_gather(0, r0, gs0)

        @pl.loop(0, n_chunks // 2)
        def pair_loop(gg):
            g = gg * 2
            fire_gather(g + 1, r1, gs1)
            drain_gather(r0, gs0)

            @pl.when(gg > 0)
            def _wait_store0():
                drain_store(o0, ss0)
            compute(g, r0, o0)
            pltpu.async_copy(o0, out_hbm.at[pl.ds(w_base + g * C, C)], ss0)

            @pl.when(gg + 1 < n_chunks // 2)
            def _fire_next():
                fire_gather(g + 2, r0, gs0)
            drain_gather(r1, gs1)

            @pl.when(gg > 0)
            def _wait_store1():
                drain_store(o1, ss1)
            compute(g + 1, r1, o1)
            pltpu.async_copy(
                o1, out_hbm.at[pl.ds(w_base + (g + 1) * C, C)], ss1)

        drain_store(o0, ss0)
        drain_store(o1, ss1)

    out = body(x1, table128, pe1)
    return out.reshape(B, S, D)


# speedup vs baseline: 2.2204x; 1.4440x over previous
import functools
import math

import jax
import jax.numpy as jnp
from jax import lax
from jax.experimental import pallas as pl
from jax.experimental.pallas import tpu as pltpu
from jax.experimental.pallas import tpu_sc as plsc

D = 64
NC, NS, L = 2, 16, 16
NW = NC * NS


def kernel(x, table, pe):
    B, S = x.shape
    N = B * S
    n_per_w = N // NW
    C = 128
    n_chunks = n_per_w // C
    spc = S // C

    x1 = x.reshape(N).astype(jnp.int32)
    pe1 = pe.astype(jnp.float32).reshape(S * D)
    table128 = jnp.pad(table, ((0, 0), (0, 128 - D)))
    scale = jnp.float32(math.sqrt(D))

    mesh = plsc.VectorSubcoreMesh(
        core_axis_name="c", subcore_axis_name="s",
        num_cores=NC, num_subcores=NS)

    @functools.partial(
        pl.kernel,
        out_type=jax.ShapeDtypeStruct((B, D, S), jnp.float32),
        mesh=mesh,
        compiler_params=pltpu.CompilerParams(needs_layout_passes=False),
        scratch_types=[
            pltpu.VMEM((S * D,), jnp.float32),
            pltpu.VMEM((n_per_w,), jnp.int32),
            pltpu.VMEM((C, 128), jnp.float32),
            pltpu.VMEM((C, 128), jnp.float32),
            pltpu.VMEM((D, C), jnp.float32),
            pltpu.VMEM((D, C), jnp.float32),
            pltpu.VMEM((D, C), jnp.float32),
            pltpu.SemaphoreType.DMA,
            pltpu.SemaphoreType.DMA,
            pltpu.SemaphoreType.DMA,
            pltpu.SemaphoreType.DMA,
        ],
    )
    def body(x_hbm, table_hbm, pe_hbm, out_hbm, pe_v, idx_v,
             r0, r1, tbuf, o0, o1, gs0, gs1, ss0, ss1):
        wid = lax.axis_index("s") * NC + lax.axis_index("c")
        w_base = wid * n_per_w
        w_seq = wid * (n_per_w // S)
        pltpu.sync_copy(pe_hbm, pe_v)
        pltpu.sync_copy(x_hbm.at[pl.ds(w_base, n_per_w)], idx_v)

        def fire_gather(g, rbuf, gsem):
            pltpu.async_copy(
                table_hbm.at[idx_v.at[pl.ds(g * C, C)]], rbuf, gsem)

        def drain_gather(rbuf, gsem):
            pltpu.make_async_copy(
                table_hbm.at[pl.ds(0, C)], rbuf, gsem).wait()

        def drain_store(obuf, ssem):
            pltpu.make_async_copy(
                obuf, out_hbm.at[0, :, pl.ds(0, C)], ssem).wait()

        def store_dst(g):
            return out_hbm.at[w_seq + lax.div(g, spc), :,
                              pl.ds(lax.rem(g, spc) * C, C)]

        def compute(g, rbuf, obuf):
            po = lax.rem(g, spc) * (C * D)

            @plsc.parallel_loop(0, C, unroll=4)
            def row_loop(r):
                iot = lax.iota(jnp.int32, L)
                for k in range(0, D, L):
                    dv = iot + k
                    sk = (dv + r) & (C - 1)
                    v = (rbuf[r, pl.ds(k, L)] * scale
                         + pe_v[pl.ds(po + r * D + k, L)])
                    plsc.store_scatter(tbuf, [dv, sk], v)

            @plsc.parallel_loop(0, D, unroll=4)
            def d_loop(d):
                iot = lax.iota(jnp.int32, L)
                dv = jnp.full((L,), d, jnp.int32)
                for sb in range(0, C, L):
                    sk = (iot + sb + d) & (C - 1)
                    obuf[d, pl.ds(sb, L)] = plsc.load_gather(tbuf, [dv, sk])

        fire_gather(0, r0, gs0)

        @pl.loop(0, n_chunks // 2)
        def pair_loop(gg):
            g = gg * 2
            fire_gather(g + 1, r1, gs1)
            drain_gather(r0, gs0)

            @pl.when(gg > 0)
            def _wait_store0():
                drain_store(o0, ss0)
            compute(g, r0, o0)
            pltpu.async_copy(o0, store_dst(g), ss0)

            @pl.when(gg + 1 < n_chunks // 2)
            def _fire_next():
                fire_gather(g + 2, r0, gs0)
            drain_gather(r1, gs1)

            @pl.when(gg > 0)
            def _wait_store1():
                drain_store(o1, ss1)
            compute(g + 1, r1, o1)
            pltpu.async_copy(o1, store_dst(g + 1), ss1)

        drain_store(o0, ss0)
        drain_store(o1, ss1)

    out = body(x1, table128, pe1)
    return out.transpose(0, 2, 1)
